# baseline (speedup 1.0000x reference)
import jax
import jax.numpy as jnp
from jax.experimental import pallas as pl
from jax.experimental.pallas import tpu as pltpu

_ROWS = 1024
_COLS = 27278
_BLOCK_ROWS = 64


def _masked_mse_body(o_ref, t_ref, res_ref, acc_ref, cnt_ref):
    i = pl.program_id(0)

    @pl.when(i == 0)
    def _init():
        acc_ref[0] = 0.0
        cnt_ref[0] = 0.0

    o = o_ref[...]
    t = t_ref[...]
    m = t != -1.0
    d = o - t
    sq = jnp.where(m, d * d, 0.0)
    acc_ref[0] += jnp.sum(sq)
    cnt_ref[0] += jnp.sum(m.astype(jnp.float32))

    @pl.when(i == pl.num_programs(0) - 1)
    def _fin():
        res_ref[0, 0] = acc_ref[0] / cnt_ref[0]


def kernel(output, target):
    grid = (_ROWS // _BLOCK_ROWS,)
    res = pl.pallas_call(
        _masked_mse_body,
        grid=grid,
        in_specs=[
            pl.BlockSpec((_BLOCK_ROWS, _COLS), lambda i: (i, 0)),
            pl.BlockSpec((_BLOCK_ROWS, _COLS), lambda i: (i, 0)),
        ],
        out_specs=pl.BlockSpec(memory_space=pltpu.SMEM),
        out_shape=jax.ShapeDtypeStruct((1, 1), jnp.float32),
        scratch_shapes=[
            pltpu.SMEM((1,), jnp.float32),
            pltpu.SMEM((1,), jnp.float32),
        ],
    )(output, target)
    return res.reshape(())
